# Initial kernel scaffold; baseline (speedup 1.0000x reference)
import jax
import jax.numpy as jnp
from jax.experimental import pallas as pl


def kernel(t, durations, coeffs):
    raise NotImplementedError("write your pallas kernel here")


# speedup vs baseline: 48.6501x; 48.6501x over previous
import functools

import jax
import jax.numpy as jnp
from jax import lax
from jax.experimental import pallas as pl
from jax.experimental.pallas import tpu as pltpu
from jax.experimental.pallas import tpu_sc as plsc

DEPTH_K = 32
L_K = 16
NWORKERS = 32
CHUNK = 8192


def _dyn_gather(x, idx):
    dnums = lax.GatherDimensionNumbers(
        offset_dims=(), collapsed_slice_dims=(0,), start_index_map=(0,))
    return lax.gather(x, idx[:, None], dnums, (1,),
                      mode=lax.GatherScatterMode.PROMISE_IN_BOUNDS)


def _vreg_cumsum(x):
    iota = lax.iota(jnp.int32, L_K)
    for sh in (1, 2, 4, 8):
        g = _dyn_gather(x, jnp.maximum(iota - sh, 0))
        x = x + jnp.where(iota >= sh, g, jnp.float32(0.0))
    return x


def _vreg_last(x):
    return _dyn_gather(x, jnp.full((L_K,), L_K - 1, jnp.int32))


def _build_tables(dur_v, coef_v, s_ref, tt_ref, y_ref, c_refs):
    iota = lax.iota(jnp.int32, L_K)
    d_lo = dur_v[pl.ds(0, L_K)]
    d_hi = dur_v[pl.ds(L_K, L_K)]
    dsq_lo = d_lo * d_lo
    dsq_hi = d_hi * d_hi
    cum_lo = _vreg_cumsum(dsq_lo)
    cum_hi = _vreg_cumsum(dsq_hi) + _vreg_last(cum_lo)

    c_lo = []
    c_hi = []
    for j in range(4):
        c_lo.append(coef_v[pl.ds(j * DEPTH_K, L_K)])
        c_hi.append(coef_v[pl.ds(j * DEPTH_K + L_K, L_K)])

    ts_lo = cum_lo - 1e-8
    ts_hi = cum_hi - 1e-8
    p_lo = ((c_lo[0] * ts_lo + c_lo[1]) * ts_lo + c_lo[2]) * ts_lo + c_lo[3]
    p_hi = ((c_hi[0] * ts_hi + c_hi[1]) * ts_hi + c_hi[2]) * ts_hi + c_hi[3]
    term_lo = p_lo * dsq_lo
    term_hi = p_hi * dsq_hi
    ys_lo = _vreg_cumsum(term_lo)
    ys_hi = _vreg_cumsum(term_hi) + _vreg_last(ys_lo)

    inf = jnp.float32(jnp.inf)
    s_hi = jnp.where(iota == (L_K - 1), inf, cum_hi)
    s_ref[pl.ds(0, L_K)] = cum_lo
    s_ref[pl.ds(L_K, L_K)] = s_hi

    shift_idx = jnp.maximum(iota - 1, 0)
    tt_lo = jnp.where(iota == 0, jnp.float32(-1e-8),
                      _dyn_gather(cum_lo, shift_idx))
    tt_hi = jnp.where(iota == 0, _vreg_last(cum_lo),
                      _dyn_gather(cum_hi, shift_idx))
    tt_ref[pl.ds(0, L_K)] = tt_lo
    tt_ref[pl.ds(L_K, L_K)] = tt_hi
    y_lo = jnp.where(iota == 0, jnp.float32(0.0),
                     _dyn_gather(ys_lo, shift_idx))
    y_hi = jnp.where(iota == 0, _vreg_last(ys_lo),
                     _dyn_gather(ys_hi, shift_idx))
    y_ref[pl.ds(0, L_K)] = y_lo
    y_ref[pl.ds(L_K, L_K)] = y_hi

    for j in range(4):
        c_refs[j][pl.ds(0, L_K)] = c_lo[j]
        c_refs[j][pl.ds(L_K, L_K)] = c_hi[j]


def _tile_body(t_hbm, dur_hbm, coef_hbm, out_hbm,
               tin, tout, dur_v, coef_v, s_ref, tt_ref, y_ref,
               c0_ref, c1_ref, c2_ref, c3_ref):
    n = t_hbm.shape[0]
    per_tile = n // NWORKERS
    n_chunks = per_tile // CHUNK
    wid = lax.axis_index("s") * 2 + lax.axis_index("c")
    base = wid * per_tile

    pltpu.sync_copy(dur_hbm, dur_v)
    pltpu.sync_copy(coef_hbm, coef_v)
    c_refs = [c0_ref, c1_ref, c2_ref, c3_ref]
    _build_tables(dur_v, coef_v, s_ref, tt_ref, y_ref, c_refs)

    def chunk_body(g, _):
        off = base + g * CHUNK
        pltpu.sync_copy(t_hbm.at[pl.ds(off, CHUNK)], tin)

        def vec_body(r, _):
            tv = tin[pl.ds(r * L_K, L_K)]
            pos = jnp.zeros((L_K,), jnp.int32)
            for s in (16, 8, 4, 2, 1):
                probe = plsc.load_gather(s_ref, [pos + (s - 1)])
                pos = jnp.where(probe < tv, pos + s, pos)
            c0 = plsc.load_gather(c0_ref, [pos])
            c1 = plsc.load_gather(c1_ref, [pos])
            c2 = plsc.load_gather(c2_ref, [pos])
            c3 = plsc.load_gather(c3_ref, [pos])
            tt = plsc.load_gather(tt_ref, [pos])
            yy = plsc.load_gather(y_ref, [pos])
            p = ((c0 * tv + c1) * tv + c2) * tv + c3
            tout[pl.ds(r * L_K, L_K)] = p * (tv - tt) + yy
            return ()

        lax.fori_loop(0, CHUNK // L_K, vec_body, ())
        pltpu.sync_copy(tout, out_hbm.at[pl.ds(off, CHUNK)])
        return ()

    lax.fori_loop(0, n_chunks, chunk_body, ())


def kernel(t, durations, coeffs):
    n = t.shape[0]
    assert n % (NWORKERS * CHUNK) == 0
    coef_flat = jnp.transpose(coeffs).reshape(-1)

    mesh = plsc.VectorSubcoreMesh(core_axis_name="c", subcore_axis_name="s")
    run = pl.kernel(
        _tile_body,
        out_type=jax.ShapeDtypeStruct((n,), jnp.float32),
        mesh=mesh,
        compiler_params=pltpu.CompilerParams(needs_layout_passes=False),
        scratch_types=[
            pltpu.VMEM((CHUNK,), jnp.float32),
            pltpu.VMEM((CHUNK,), jnp.float32),
            pltpu.VMEM((DEPTH_K,), jnp.float32),
            pltpu.VMEM((4 * DEPTH_K,), jnp.float32),
            pltpu.VMEM((DEPTH_K,), jnp.float32),
            pltpu.VMEM((DEPTH_K,), jnp.float32),
            pltpu.VMEM((DEPTH_K,), jnp.float32),
            pltpu.VMEM((DEPTH_K,), jnp.float32),
            pltpu.VMEM((DEPTH_K,), jnp.float32),
            pltpu.VMEM((DEPTH_K,), jnp.float32),
            pltpu.VMEM((DEPTH_K,), jnp.float32),
        ],
    )
    return run(t, durations, coef_flat)


# speedup vs baseline: 250.5213x; 5.1494x over previous
import functools

import jax
import jax.numpy as jnp
from jax import lax
from jax.experimental import pallas as pl
from jax.experimental.pallas import tpu as pltpu
from jax.experimental.pallas import tpu_sc as plsc

DEPTH_K = 32
L_K = 16
NWORKERS = 32
CHUNK = 8192


def _dyn_gather(x, idx):
    dnums = lax.GatherDimensionNumbers(
        offset_dims=(), collapsed_slice_dims=(0,), start_index_map=(0,))
    return lax.gather(x, idx[:, None], dnums, (1,),
                      mode=lax.GatherScatterMode.PROMISE_IN_BOUNDS)


def _vreg_cumsum(x):
    iota = lax.iota(jnp.int32, L_K)
    for sh in (1, 2, 4, 8):
        g = _dyn_gather(x, jnp.maximum(iota - sh, 0))
        x = x + jnp.where(iota >= sh, g, jnp.float32(0.0))
    return x


def _vreg_last(x):
    return _dyn_gather(x, jnp.full((L_K,), L_K - 1, jnp.int32))


def _build_tables(dur_v, coef_v, s_ref, tt_ref, y_ref, c_refs):
    iota = lax.iota(jnp.int32, L_K)
    d_lo = dur_v[pl.ds(0, L_K)]
    d_hi = dur_v[pl.ds(L_K, L_K)]
    dsq_lo = d_lo * d_lo
    dsq_hi = d_hi * d_hi
    cum_lo = _vreg_cumsum(dsq_lo)
    cum_hi = _vreg_cumsum(dsq_hi) + _vreg_last(cum_lo)

    c_lo = []
    c_hi = []
    for j in range(4):
        c_lo.append(coef_v[pl.ds(j * DEPTH_K, L_K)])
        c_hi.append(coef_v[pl.ds(j * DEPTH_K + L_K, L_K)])

    ts_lo = cum_lo - 1e-8
    ts_hi = cum_hi - 1e-8
    p_lo = ((c_lo[0] * ts_lo + c_lo[1]) * ts_lo + c_lo[2]) * ts_lo + c_lo[3]
    p_hi = ((c_hi[0] * ts_hi + c_hi[1]) * ts_hi + c_hi[2]) * ts_hi + c_hi[3]
    term_lo = p_lo * dsq_lo
    term_hi = p_hi * dsq_hi
    ys_lo = _vreg_cumsum(term_lo)
    ys_hi = _vreg_cumsum(term_hi) + _vreg_last(ys_lo)

    inf = jnp.float32(jnp.inf)
    s_hi = jnp.where(iota == (L_K - 1), inf, cum_hi)
    s_ref[pl.ds(0, L_K)] = cum_lo
    s_ref[pl.ds(L_K, L_K)] = s_hi

    shift_idx = jnp.maximum(iota - 1, 0)
    tt_lo = jnp.where(iota == 0, jnp.float32(-1e-8),
                      _dyn_gather(cum_lo, shift_idx))
    tt_hi = jnp.where(iota == 0, _vreg_last(cum_lo),
                      _dyn_gather(cum_hi, shift_idx))
    tt_ref[pl.ds(0, L_K)] = tt_lo
    tt_ref[pl.ds(L_K, L_K)] = tt_hi
    y_lo = jnp.where(iota == 0, jnp.float32(0.0),
                     _dyn_gather(ys_lo, shift_idx))
    y_hi = jnp.where(iota == 0, _vreg_last(ys_lo),
                     _dyn_gather(ys_hi, shift_idx))
    y_ref[pl.ds(0, L_K)] = y_lo
    y_ref[pl.ds(L_K, L_K)] = y_hi

    for j in range(4):
        c_refs[j][pl.ds(0, L_K)] = c_lo[j]
        c_refs[j][pl.ds(L_K, L_K)] = c_hi[j]


def _tile_body(t_hbm, dur_hbm, coef_hbm, out_hbm,
               tin, tout, dur_v, coef_v, s_ref, tt_ref, y_ref,
               c0_ref, c1_ref, c2_ref, c3_ref):
    n = t_hbm.shape[0]
    per_tile = n // NWORKERS
    n_chunks = per_tile // CHUNK
    wid = lax.axis_index("s") * 2 + lax.axis_index("c")
    base = wid * per_tile

    pltpu.sync_copy(dur_hbm, dur_v)
    pltpu.sync_copy(coef_hbm, coef_v)
    c_refs = [c0_ref, c1_ref, c2_ref, c3_ref]
    _build_tables(dur_v, coef_v, s_ref, tt_ref, y_ref, c_refs)

    def chunk_body(g, _):
        off = base + g * CHUNK
        pltpu.sync_copy(t_hbm.at[pl.ds(off, CHUNK)], tin)

        @plsc.parallel_loop(0, CHUNK, L_K, unroll=8)
        def vec_body(off_r):
            tv = tin[pl.ds(off_r, L_K)]
            pos = jnp.zeros((L_K,), jnp.int32)
            for s in (16, 8, 4, 2, 1):
                probe = plsc.load_gather(s_ref, [pos + (s - 1)])
                pos = jnp.where(probe < tv, pos + s, pos)
            c0 = plsc.load_gather(c0_ref, [pos])
            c1 = plsc.load_gather(c1_ref, [pos])
            c2 = plsc.load_gather(c2_ref, [pos])
            c3 = plsc.load_gather(c3_ref, [pos])
            tt = plsc.load_gather(tt_ref, [pos])
            yy = plsc.load_gather(y_ref, [pos])
            p = ((c0 * tv + c1) * tv + c2) * tv + c3
            tout[pl.ds(off_r, L_K)] = p * (tv - tt) + yy

        pltpu.sync_copy(tout, out_hbm.at[pl.ds(off, CHUNK)])
        return ()

    lax.fori_loop(0, n_chunks, chunk_body, ())


def kernel(t, durations, coeffs):
    n = t.shape[0]
    assert n % (NWORKERS * CHUNK) == 0
    coef_flat = jnp.transpose(coeffs).reshape(-1)

    mesh = plsc.VectorSubcoreMesh(core_axis_name="c", subcore_axis_name="s")
    run = pl.kernel(
        _tile_body,
        out_type=jax.ShapeDtypeStruct((n,), jnp.float32),
        mesh=mesh,
        compiler_params=pltpu.CompilerParams(needs_layout_passes=False),
        scratch_types=[
            pltpu.VMEM((CHUNK,), jnp.float32),
            pltpu.VMEM((CHUNK,), jnp.float32),
            pltpu.VMEM((DEPTH_K,), jnp.float32),
            pltpu.VMEM((4 * DEPTH_K,), jnp.float32),
            pltpu.VMEM((DEPTH_K,), jnp.float32),
            pltpu.VMEM((DEPTH_K,), jnp.float32),
            pltpu.VMEM((DEPTH_K,), jnp.float32),
            pltpu.VMEM((DEPTH_K,), jnp.float32),
            pltpu.VMEM((DEPTH_K,), jnp.float32),
            pltpu.VMEM((DEPTH_K,), jnp.float32),
            pltpu.VMEM((DEPTH_K,), jnp.float32),
        ],
    )
    return run(t, durations, coef_flat)


# speedup vs baseline: 682.9998x; 2.7263x over previous
import jax
import jax.numpy as jnp
from jax import lax
from jax.experimental import pallas as pl
from jax.experimental.pallas import tpu as pltpu
from jax.experimental.pallas import tpu_sc as plsc

DEPTH_K = 32
L_K = 16
NWORKERS = 32
CHUNK = 8192


def _dyn_gather(x, idx):
    dnums = lax.GatherDimensionNumbers(
        offset_dims=(), collapsed_slice_dims=(0,), start_index_map=(0,))
    return lax.gather(x, idx[:, None], dnums, (1,),
                      mode=lax.GatherScatterMode.PROMISE_IN_BOUNDS)


def _vreg_cumsum(x):
    iota = lax.iota(jnp.int32, L_K)
    for sh in (1, 2, 4, 8):
        g = _dyn_gather(x, jnp.maximum(iota - sh, 0))
        x = x + jnp.where(iota >= sh, g, jnp.float32(0.0))
    return x


def _vreg_cumsum_i32(x):
    iota = lax.iota(jnp.int32, L_K)
    for sh in (1, 2, 4, 8):
        g = _dyn_gather(x, jnp.maximum(iota - sh, 0))
        x = x + jnp.where(iota >= sh, g, 0)
    return x


def _vreg_last(x):
    return _dyn_gather(x, jnp.full((L_K,), L_K - 1, jnp.int32))


def _bcast(x, lane):
    return _dyn_gather(x, jnp.full((L_K,), lane, jnp.int32))


def _build_tables(dur_v, coef_v, s_ref, q_refs):
    iota = lax.iota(jnp.int32, L_K)
    d_lo = dur_v[pl.ds(0, L_K)]
    d_hi = dur_v[pl.ds(L_K, L_K)]
    dsq_lo = d_lo * d_lo
    dsq_hi = d_hi * d_hi
    cum_lo = _vreg_cumsum(dsq_lo)
    cum_hi = _vreg_cumsum(dsq_hi) + _vreg_last(cum_lo)

    c_lo = []
    c_hi = []
    for j in range(4):
        c_lo.append(coef_v[pl.ds(j * DEPTH_K, L_K)])
        c_hi.append(coef_v[pl.ds(j * DEPTH_K + L_K, L_K)])

    ts_lo = cum_lo - 1e-8
    ts_hi = cum_hi - 1e-8
    p_lo = ((c_lo[0] * ts_lo + c_lo[1]) * ts_lo + c_lo[2]) * ts_lo + c_lo[3]
    p_hi = ((c_hi[0] * ts_hi + c_hi[1]) * ts_hi + c_hi[2]) * ts_hi + c_hi[3]
    ys_lo = _vreg_cumsum(p_lo * dsq_lo)
    ys_hi = _vreg_cumsum(p_hi * dsq_hi) + _vreg_last(ys_lo)

    inf = jnp.float32(jnp.inf)
    s_hi = jnp.where(iota == (L_K - 1), inf, cum_hi)
    s_ref[pl.ds(0, L_K)] = cum_lo
    s_ref[pl.ds(L_K, L_K)] = s_hi

    shift_idx = jnp.maximum(iota - 1, 0)
    tt_lo = jnp.where(iota == 0, jnp.float32(-1e-8),
                      _dyn_gather(cum_lo, shift_idx))
    tt_hi = jnp.where(iota == 0, _vreg_last(cum_lo),
                      _dyn_gather(cum_hi, shift_idx))
    y_lo = jnp.where(iota == 0, jnp.float32(0.0),
                     _dyn_gather(ys_lo, shift_idx))
    y_hi = jnp.where(iota == 0, _vreg_last(ys_lo),
                     _dyn_gather(ys_hi, shift_idx))

    q_lo = [c_lo[0],
            c_lo[1] - c_lo[0] * tt_lo,
            c_lo[2] - c_lo[1] * tt_lo,
            c_lo[3] - c_lo[2] * tt_lo,
            y_lo - c_lo[3] * tt_lo]
    q_hi = [c_hi[0],
            c_hi[1] - c_hi[0] * tt_hi,
            c_hi[2] - c_hi[1] * tt_hi,
            c_hi[3] - c_hi[2] * tt_hi,
            y_hi - c_hi[3] * tt_hi]
    for j in range(5):
        q_refs[j][pl.ds(0, L_K)] = q_lo[j]
        q_refs[j][pl.ds(L_K, L_K)] = q_hi[j]

    ones = jnp.float32(1.0)
    nlt_lo = jnp.where(cum_lo < ones, 1, 0)
    nlt_hi = jnp.where(s_hi < ones, 1, 0)
    kmax_splat = (_vreg_last(_vreg_cumsum_i32(nlt_lo))
                  + _vreg_last(_vreg_cumsum_i32(nlt_hi)))

    return kmax_splat


def _tile_body(t_hbm, dur_hbm, coef_hbm, out_hbm,
               tin, tout, dur_v, coef_v, s_ref,
               q0_ref, q1_ref, q2_ref, q3_ref, q4_ref, ksc_ref,
               in_sem0, in_sem1, out_sem0, out_sem1):
    n = t_hbm.shape[0]
    per_tile = n // NWORKERS
    n_chunks = per_tile // CHUNK
    wid = lax.axis_index("s") * 2 + lax.axis_index("c")
    base = wid * per_tile

    in_sems = (in_sem0, in_sem1)
    out_sems = (out_sem0, out_sem1)
    pltpu.sync_copy(dur_hbm, dur_v)
    pltpu.sync_copy(coef_hbm, coef_v)
    kmax_splat = _build_tables(
        dur_v, coef_v, s_ref, [q0_ref, q1_ref, q2_ref, q3_ref, q4_ref])
    ksc_ref[pl.ds(0, L_K)] = kmax_splat
    kmax = ksc_ref[pl.ds(0, L_K)][0]

    for b in range(2):
        pltpu.async_copy(t_hbm.at[pl.ds(base + b * CHUNK, CHUNK)],
                         tin.at[b], in_sems[b])

    def chunk_pair(gp, _):
      for b in range(2):
        c = gp * 2 + b
        off = base + c * CHUNK
        pltpu.make_async_copy(t_hbm.at[pl.ds(off, CHUNK)], tin.at[b],
                              in_sems[b]).wait()

        @pl.when(c >= 2)
        def _wait_out():
            pltpu.make_async_copy(tout.at[b], out_hbm.at[pl.ds(off, CHUNK)],
                                  out_sems[b]).wait()

        @pl.when(kmax == 0)
        def _const():
            c0 = _bcast(q0_ref[pl.ds(0, L_K)], 0)
            c1 = _bcast(q1_ref[pl.ds(0, L_K)], 0)
            c2 = _bcast(q2_ref[pl.ds(0, L_K)], 0)
            c3 = _bcast(q3_ref[pl.ds(0, L_K)], 0)
            c4 = _bcast(q4_ref[pl.ds(0, L_K)], 0)

            @plsc.parallel_loop(0, CHUNK, L_K, unroll=8)
            def vec_body(off_r):
                tv = tin[b, pl.ds(off_r, L_K)]
                r = (((c0 * tv + c1) * tv + c2) * tv + c3) * tv + c4
                tout[b, pl.ds(off_r, L_K)] = r

        @pl.when((kmax >= 1) & (kmax <= 4))
        def _fast():
            s_lo = s_ref[pl.ds(0, L_K)]
            b0 = _bcast(s_lo, 0)
            b1 = _bcast(s_lo, 1)
            b2 = _bcast(s_lo, 2)
            b3 = _bcast(s_lo, 3)

            @plsc.parallel_loop(0, CHUNK, L_K, unroll=6)
            def vec_body(off_r):
                tv = tin[b, pl.ds(off_r, L_K)]
                one = jnp.int32(1)
                zero = jnp.int32(0)
                pos = (jnp.where(b0 < tv, one, zero)
                       + jnp.where(b1 < tv, one, zero)
                       + jnp.where(b2 < tv, one, zero)
                       + jnp.where(b3 < tv, one, zero))
                q0 = plsc.load_gather(q0_ref, [pos])
                q1 = plsc.load_gather(q1_ref, [pos])
                q2 = plsc.load_gather(q2_ref, [pos])
                q3 = plsc.load_gather(q3_ref, [pos])
                q4 = plsc.load_gather(q4_ref, [pos])
                r = (((q0 * tv + q1) * tv + q2) * tv + q3) * tv + q4
                tout[b, pl.ds(off_r, L_K)] = r

        @pl.when(kmax > 4)
        def _general():
            s_lo = s_ref[pl.ds(0, L_K)]
            s_hi2 = s_ref[pl.ds(L_K, L_K)]
            s15 = _vreg_last(s_lo)
            s7 = _bcast(s_lo, 7)
            s23 = _bcast(s_hi2, 7)

            @plsc.parallel_loop(0, CHUNK, L_K, unroll=8)
            def vec_body(off_r):
                tv = tin[b, pl.ds(off_r, L_K)]
                m1 = s15 < tv
                pos = jnp.where(m1, jnp.int32(16), jnp.int32(0))
                probe2 = jnp.where(m1, s23, s7)
                pos = jnp.where(probe2 < tv, pos + 8, pos)
                for s in (4, 2, 1):
                    probe = plsc.load_gather(s_ref, [pos + (s - 1)])
                    pos = jnp.where(probe < tv, pos + s, pos)
                q0 = plsc.load_gather(q0_ref, [pos])
                q1 = plsc.load_gather(q1_ref, [pos])
                q2 = plsc.load_gather(q2_ref, [pos])
                q3 = plsc.load_gather(q3_ref, [pos])
                q4 = plsc.load_gather(q4_ref, [pos])
                r = (((q0 * tv + q1) * tv + q2) * tv + q3) * tv + q4
                tout[b, pl.ds(off_r, L_K)] = r

        pltpu.async_copy(tout.at[b], out_hbm.at[pl.ds(off, CHUNK)],
                         out_sems[b])

        @pl.when(c + 2 < n_chunks)
        def _prefetch():
            pltpu.async_copy(t_hbm.at[pl.ds(off + 2 * CHUNK, CHUNK)],
                             tin.at[b], in_sems[b])
      return ()

    lax.fori_loop(0, n_chunks // 2, chunk_pair, ())
    for b in range(2):
        pltpu.make_async_copy(tout.at[b], out_hbm.at[pl.ds(base, CHUNK)],
                              out_sems[b]).wait()


def kernel(t, durations, coeffs):
    n = t.shape[0]
    assert n % (NWORKERS * CHUNK) == 0
    coef_flat = jnp.transpose(coeffs).reshape(-1)

    mesh = plsc.VectorSubcoreMesh(core_axis_name="c", subcore_axis_name="s")
    run = pl.kernel(
        _tile_body,
        out_type=jax.ShapeDtypeStruct((n,), jnp.float32),
        mesh=mesh,
        compiler_params=pltpu.CompilerParams(needs_layout_passes=False),
        scratch_types=[
            pltpu.VMEM((2, CHUNK), jnp.float32),
            pltpu.VMEM((2, CHUNK), jnp.float32),
            pltpu.VMEM((DEPTH_K,), jnp.float32),
            pltpu.VMEM((4 * DEPTH_K,), jnp.float32),
            pltpu.VMEM((DEPTH_K,), jnp.float32),
            pltpu.VMEM((DEPTH_K,), jnp.float32),
            pltpu.VMEM((DEPTH_K,), jnp.float32),
            pltpu.VMEM((DEPTH_K,), jnp.float32),
            pltpu.VMEM((DEPTH_K,), jnp.float32),
            pltpu.VMEM((DEPTH_K,), jnp.float32),
            pltpu.VMEM((L_K,), jnp.int32),
            pltpu.SemaphoreType.DMA,
            pltpu.SemaphoreType.DMA,
            pltpu.SemaphoreType.DMA,
            pltpu.SemaphoreType.DMA,
        ],
    )
    return run(t, durations, coef_flat)
